# baseline (speedup 1.0000x reference)
import functools
import math

import jax
import jax.numpy as jnp
from jax import lax
from jax.experimental import pallas as pl
from jax.experimental.pallas import tpu as pltpu
from jax.experimental.pallas import tpu_sc as plsc

N = 10000
E = 320000
D = 128
L = 3

NC = 2
NS = 16
NW = NC * NS
EW = E // NW
CB = 80
NCHB = EW // CB
IB = 25
NP = 10240
RPT = NP // NS
WB = 80
NWB = RPT // WB
NPT = NP // NS

_SC_MESH = plsc.VectorSubcoreMesh(core_axis_name="c", subcore_axis_name="s")


def _qkvs_body(h_ref, wq_ref, wk_ref, wv_ref, ws_ref, bq_ref, bk_ref, bv_ref,
               bs_ref, q_ref, kv_ref, s_ref):
    h = h_ref[...]
    q_ref[...] = h @ wq_ref[...] + bq_ref[...]
    k = h @ wk_ref[...] + bk_ref[...]
    v = h @ wv_ref[...] + bv_ref[...]
    kv_ref[:, :D] = k
    kv_ref[:, D:] = v
    s_ref[...] = h @ ws_ref[...] + bs_ref[...]


def _tc_qkvs(h, Wq, Wk, Wv, Ws, bq, bk, bv, bs):
    R = 400
    grid = (N // R,)
    row = pl.BlockSpec((R, D), lambda i: (i, 0))
    full = pl.BlockSpec((D, D), lambda i: (0, 0))
    bias = pl.BlockSpec((1, D), lambda i: (0, 0))
    return pl.pallas_call(
        _qkvs_body,
        grid=grid,
        in_specs=[row, full, full, full, full, bias, bias, bias, bias],
        out_specs=[row, pl.BlockSpec((R, 2 * D), lambda i: (i, 0)), row],
        out_shape=[
            jax.ShapeDtypeStruct((N, D), jnp.float32),
            jax.ShapeDtypeStruct((N, 2 * D), jnp.float32),
            jax.ShapeDtypeStruct((N, D), jnp.float32),
        ],
    )(h, Wq, Wk, Wv, Ws, bq.reshape(1, D), bk.reshape(1, D),
      bv.reshape(1, D), bs.reshape(1, D))


def _fused_body(q_hbm, kv_hbm, dst_hbm, src_hbm, zeros_hbm, u_out, d_out,
                idxA, srcA, qA, kvA,
                wvbuf, ebuf, dbuf, shared, shared_d, semA):
    c = lax.axis_index("c")
    s = lax.axis_index("s")
    wid = s * NC + c
    base = wid * EW

    pltpu.sync_copy(zeros_hbm, wvbuf)

    def zchunk(t, carry):
        pltpu.sync_copy(wvbuf, shared.at[pl.ds(s * RPT + t * WB, WB)])
        return carry

    lax.fori_loop(0, NWB, zchunk, 0)

    def zd(r, carry):
        dbuf[pl.ds(r * 16, 16)] = jnp.zeros((16,), jnp.float32)
        return carry

    lax.fori_loop(0, NPT // 16, zd, 0)
    pltpu.sync_copy(dbuf, shared_d.at[pl.ds(s * NPT, NPT)])
    plsc.subcore_barrier()

    inv = 1.0 / math.sqrt(D)
    lanes = lax.iota(jnp.int32, 16)
    x8 = jnp.bitwise_xor(lanes, 8)
    x4 = jnp.bitwise_xor(lanes, 4)
    x2 = jnp.bitwise_xor(lanes, 2)
    x1 = jnp.bitwise_xor(lanes, 1)

    def drain(qbuf, kvbuf, sem_):
        pltpu.make_async_copy(q_hbm.at[pl.ds(0, CB)], qbuf, sem_).wait()
        pltpu.make_async_copy(kv_hbm.at[pl.ds(0, CB)], kvbuf, sem_).wait()

    def compute_scatter(idxb, qbuf, kvbuf):
        @plsc.parallel_loop(0, CB // 16, 1, unroll=2)
        def group(g):
            egroup = jnp.zeros((16,), jnp.float32)
            for ii in range(16):
                i = g * 16 + ii
                a0 = qbuf[i, pl.ds(0, 16)] * kvbuf[i, pl.ds(0, 16)]
                a1 = qbuf[i, pl.ds(16, 16)] * kvbuf[i, pl.ds(16, 16)]
                a2 = qbuf[i, pl.ds(32, 16)] * kvbuf[i, pl.ds(32, 16)]
                a3 = qbuf[i, pl.ds(48, 16)] * kvbuf[i, pl.ds(48, 16)]
                a0 = a0 + qbuf[i, pl.ds(64, 16)] * kvbuf[i, pl.ds(64, 16)]
                a1 = a1 + qbuf[i, pl.ds(80, 16)] * kvbuf[i, pl.ds(80, 16)]
                a2 = a2 + qbuf[i, pl.ds(96, 16)] * kvbuf[i, pl.ds(96, 16)]
                a3 = a3 + qbuf[i, pl.ds(112, 16)] * kvbuf[i, pl.ds(112, 16)]
                t = (a0 + a1) + (a2 + a3)
                t = t + t[x8]
                t = t + t[x4]
                t = t + t[x2]
                t = t + t[x1]
                evec = jnp.exp(t * inv)
                egroup = jnp.where(lanes == ii, evec, egroup)
                for dc in range(D // 16):
                    wvbuf[i, pl.ds(dc * 16, 16)] = (
                        kvbuf[i, pl.ds(D + dc * 16, 16)] * evec)
            ebuf[pl.ds(g * 16, 16)] = egroup

        pltpu.sync_copy(wvbuf, shared.at[idxb], add=True)
        pltpu.sync_copy(ebuf, shared_d.at[idxb], add=True)

    def bigchunk(b, carry):
        off = base + b * (IB * CB)
        pltpu.sync_copy(dst_hbm.at[pl.ds(off, IB * CB)], idxA)
        pltpu.sync_copy(src_hbm.at[pl.ds(off, IB * CB)], srcA)
        def sub(u, carry2):
            iv = idxA.at[pl.ds(u * CB, CB)]
            sv = srcA.at[pl.ds(u * CB, CB)]
            pltpu.async_copy(q_hbm.at[iv], qA, semA)
            pltpu.async_copy(kv_hbm.at[sv], kvA, semA)
            drain(qA, kvA, semA)
            compute_scatter(iv, qA, kvA)
            return carry2

        lax.fori_loop(0, IB, sub, 0)
        return carry

    lax.fori_loop(0, NCHB // IB, bigchunk, 0)
    plsc.subcore_barrier()

    def wchunk(t, carry):
        r0 = s * RPT + t * WB
        pltpu.sync_copy(shared.at[pl.ds(r0, WB)], wvbuf)
        pltpu.sync_copy(wvbuf, u_out.at[c].at[pl.ds(r0, WB)])
        return carry

    lax.fori_loop(0, NWB, wchunk, 0)
    pltpu.sync_copy(shared_d.at[pl.ds(s * NPT, NPT)], dbuf)
    pltpu.sync_copy(dbuf, d_out.at[c].at[pl.ds(s * NPT, NPT)])


def _sc_fused(q, kv, dst, src, zeros_wb):
    f = pl.kernel(
        _fused_body,
        out_type=[
            jax.ShapeDtypeStruct((NC, NP, D), jnp.float32),
            jax.ShapeDtypeStruct((NC, NP), jnp.float32),
        ],
        mesh=_SC_MESH,
        scratch_types=[
            pltpu.VMEM((IB * CB,), jnp.int32),
            pltpu.VMEM((IB * CB,), jnp.int32),
            pltpu.VMEM((CB, D), jnp.float32),
            pltpu.VMEM((CB, 2 * D), jnp.float32),
            pltpu.VMEM((CB, D), jnp.float32),
            pltpu.VMEM((CB,), jnp.float32),
            pltpu.VMEM((NPT,), jnp.float32),
            pltpu.VMEM_SHARED((NP, D), jnp.float32),
            pltpu.VMEM_SHARED((NP,), jnp.float32),
            pltpu.SemaphoreType.DMA,
        ],
    )
    return f(q, kv, dst, src, zeros_wb)


def _final_body(is_first, u0_ref, u1_ref, d0_ref, d1_ref, s_ref, hi_ref,
                g_ref, b_ref, out_ref, hi_out_ref):
    us = u0_ref[...] + u1_ref[...]
    den = d0_ref[...] + d1_ref[...]
    agg = us / (den + 1e-16)
    t = jax.nn.relu(agg + s_ref[...])
    if is_first:
        hi_out_ref[...] = t
    else:
        t = t + hi_ref[...]
        hi_out_ref[...] = hi_ref[...]
    mu = jnp.mean(t, axis=1, keepdims=True)
    var = jnp.mean(jnp.square(t - mu), axis=1, keepdims=True)
    out_ref[...] = (t - mu) * lax.rsqrt(var + 1e-5) * g_ref[...] + b_ref[...]


def _tc_final(u, den, s, h_init, gamma, beta, is_first):
    R = 400
    grid = (N // R,)
    row = pl.BlockSpec((R, D), lambda i: (i, 0))
    col = pl.BlockSpec((R, 1), lambda i: (i, 0))
    vec = pl.BlockSpec((1, D), lambda i: (0, 0))
    return pl.pallas_call(
        functools.partial(_final_body, is_first),
        grid=grid,
        in_specs=[row, row, col, col, row, row, vec, vec],
        out_specs=[row, row],
        out_shape=[
            jax.ShapeDtypeStruct((N, D), jnp.float32),
            jax.ShapeDtypeStruct((N, D), jnp.float32),
        ],
    )(u[0], u[1], den[0], den[1], s, h_init, gamma.reshape(1, D),
      beta.reshape(1, D))


def kernel(x, edge_index, Wq, bq, Wk, bk, Wv, bv, Ws, bs, gamma, beta):
    src = edge_index[0]
    dst = edge_index[1]
    zeros_wb = jnp.zeros((WB, D), jnp.float32)

    h = x
    h_init = h
    for i in range(L):
        q, kv, s = _tc_qkvs(h, Wq[i], Wk[i], Wv[i], Ws[i], bq[i], bk[i],
                            bv[i], bs[i])
        u, dacc = _sc_fused(q, kv, dst, src, zeros_wb)
        un = u[:, :N]
        den = dacc.reshape(NC, NP, 1)[:, :N]
        h, h_init = _tc_final(un, den, s, h_init, gamma[i], beta[i], i == 0)
    return h


# speedup vs baseline: 1.3384x; 1.3384x over previous
import functools
import math

import jax
import jax.numpy as jnp
from jax import lax
from jax.experimental import pallas as pl
from jax.experimental.pallas import tpu as pltpu
from jax.experimental.pallas import tpu_sc as plsc

N = 10000
E = 320000
D = 128
L = 3

NC = 2
NS = 16
NW = NC * NS
EW = E // NW
CB = 80
NCHB = EW // CB
IB = 25
NP = 10240
RPT = NP // NS
WB = 80
NWB = RPT // WB
NPT = NP // NS

_SC_MESH = plsc.VectorSubcoreMesh(core_axis_name="c", subcore_axis_name="s")


def _qkvs_body(h_ref, wq_ref, wk_ref, wv_ref, ws_ref, bq_ref, bk_ref, bv_ref,
               bs_ref, q_ref, kv_ref, s_ref):
    h = h_ref[...]
    q_ref[...] = h @ wq_ref[...] + bq_ref[...]
    k = h @ wk_ref[...] + bk_ref[...]
    v = h @ wv_ref[...] + bv_ref[...]
    kv_ref[:, :D] = k
    kv_ref[:, D:] = v
    s_ref[...] = h @ ws_ref[...] + bs_ref[...]


def _tc_qkvs(h, Wq, Wk, Wv, Ws, bq, bk, bv, bs):
    R = 400
    grid = (N // R,)
    row = pl.BlockSpec((R, D), lambda i: (i, 0))
    full = pl.BlockSpec((D, D), lambda i: (0, 0))
    bias = pl.BlockSpec((1, D), lambda i: (0, 0))
    return pl.pallas_call(
        _qkvs_body,
        grid=grid,
        in_specs=[row, full, full, full, full, bias, bias, bias, bias],
        out_specs=[row, pl.BlockSpec((R, 2 * D), lambda i: (i, 0)), row],
        out_shape=[
            jax.ShapeDtypeStruct((N, D), jnp.float32),
            jax.ShapeDtypeStruct((N, 2 * D), jnp.float32),
            jax.ShapeDtypeStruct((N, D), jnp.float32),
        ],
    )(h, Wq, Wk, Wv, Ws, bq.reshape(1, D), bk.reshape(1, D),
      bv.reshape(1, D), bs.reshape(1, D))


def _fused_body(q_hbm, kv_hbm, dst_hbm, src_hbm, zeros_hbm, u_out, d_out,
                idxA, srcA, qA, kvA,
                wvbuf, ebuf, dbuf, shared, shared_d, semA):
    c = lax.axis_index("c")
    s = lax.axis_index("s")
    wid = s * NC + c
    base = wid * EW

    pltpu.sync_copy(zeros_hbm, wvbuf)

    def zchunk(t, carry):
        pltpu.sync_copy(wvbuf, shared.at[pl.ds(s * RPT + t * WB, WB)])
        return carry

    lax.fori_loop(0, NWB, zchunk, 0)

    def zd(r, carry):
        dbuf[pl.ds(r * 16, 16)] = jnp.zeros((16,), jnp.float32)
        return carry

    lax.fori_loop(0, NPT // 16, zd, 0)
    pltpu.sync_copy(dbuf, shared_d.at[pl.ds(s * NPT, NPT)])
    plsc.subcore_barrier()

    inv = 1.0 / math.sqrt(D)
    lanes = lax.iota(jnp.int32, 16)
    x8 = jnp.bitwise_xor(lanes, 8)
    x4 = jnp.bitwise_xor(lanes, 4)
    x2 = jnp.bitwise_xor(lanes, 2)
    x1 = jnp.bitwise_xor(lanes, 1)

    def drain(qbuf, kvbuf, sem_):
        pltpu.make_async_copy(q_hbm.at[pl.ds(0, CB)], qbuf, sem_).wait()
        pltpu.make_async_copy(kv_hbm.at[pl.ds(0, CB)], kvbuf, sem_).wait()

    def compute_scatter(idxb, qbuf, kvbuf):
        @plsc.parallel_loop(0, CB // 16, 1, unroll=1)
        def group(g):
            egroup = jnp.zeros((16,), jnp.float32)
            for ii in range(16):
                i = g * 16 + ii
                a0 = qbuf[i, pl.ds(0, 16)] * kvbuf[i, pl.ds(0, 16)]
                a1 = qbuf[i, pl.ds(16, 16)] * kvbuf[i, pl.ds(16, 16)]
                a2 = qbuf[i, pl.ds(32, 16)] * kvbuf[i, pl.ds(32, 16)]
                a3 = qbuf[i, pl.ds(48, 16)] * kvbuf[i, pl.ds(48, 16)]
                a0 = a0 + qbuf[i, pl.ds(64, 16)] * kvbuf[i, pl.ds(64, 16)]
                a1 = a1 + qbuf[i, pl.ds(80, 16)] * kvbuf[i, pl.ds(80, 16)]
                a2 = a2 + qbuf[i, pl.ds(96, 16)] * kvbuf[i, pl.ds(96, 16)]
                a3 = a3 + qbuf[i, pl.ds(112, 16)] * kvbuf[i, pl.ds(112, 16)]
                t = (a0 + a1) + (a2 + a3)
                t = t + t[x8]
                t = t + t[x4]
                t = t + t[x2]
                t = t + t[x1]
                evec = jnp.exp(t * inv)
                egroup = jnp.where(lanes == ii, evec, egroup)
                for dc in range(D // 16):
                    wvbuf[i, pl.ds(dc * 16, 16)] = (
                        kvbuf[i, pl.ds(D + dc * 16, 16)] * evec)
            ebuf[pl.ds(g * 16, 16)] = egroup

        pltpu.sync_copy(wvbuf, shared.at[idxb], add=True)
        pltpu.sync_copy(ebuf, shared_d.at[idxb], add=True)

    def bigchunk(b, carry):
        off = base + b * (IB * CB)
        pltpu.sync_copy(dst_hbm.at[pl.ds(off, IB * CB)], idxA)
        pltpu.sync_copy(src_hbm.at[pl.ds(off, IB * CB)], srcA)
        def sub(u, carry2):
            iv = idxA.at[pl.ds(u * CB, CB)]
            sv = srcA.at[pl.ds(u * CB, CB)]
            pltpu.async_copy(q_hbm.at[iv], qA, semA)
            pltpu.async_copy(kv_hbm.at[sv], kvA, semA)
            drain(qA, kvA, semA)
            compute_scatter(iv, qA, kvA)
            return carry2

        lax.fori_loop(0, IB, sub, 0)
        return carry

    lax.fori_loop(0, NCHB // IB, bigchunk, 0)
    plsc.subcore_barrier()

    def wchunk(t, carry):
        r0 = s * RPT + t * WB
        pltpu.sync_copy(shared.at[pl.ds(r0, WB)], wvbuf)
        pltpu.sync_copy(wvbuf, u_out.at[c].at[pl.ds(r0, WB)])
        return carry

    lax.fori_loop(0, NWB, wchunk, 0)
    pltpu.sync_copy(shared_d.at[pl.ds(s * NPT, NPT)], dbuf)
    pltpu.sync_copy(dbuf, d_out.at[c].at[pl.ds(s * NPT, NPT)])


def _sc_fused(q, kv, dst, src, zeros_wb):
    f = pl.kernel(
        _fused_body,
        out_type=[
            jax.ShapeDtypeStruct((NC, NP, D), jnp.float32),
            jax.ShapeDtypeStruct((NC, NP), jnp.float32),
        ],
        mesh=_SC_MESH,
        scratch_types=[
            pltpu.VMEM((IB * CB,), jnp.int32),
            pltpu.VMEM((IB * CB,), jnp.int32),
            pltpu.VMEM((CB, D), jnp.float32),
            pltpu.VMEM((CB, 2 * D), jnp.float32),
            pltpu.VMEM((CB, D), jnp.float32),
            pltpu.VMEM((CB,), jnp.float32),
            pltpu.VMEM((NPT,), jnp.float32),
            pltpu.VMEM_SHARED((NP, D), jnp.float32),
            pltpu.VMEM_SHARED((NP,), jnp.float32),
            pltpu.SemaphoreType.DMA,
        ],
    )
    return f(q, kv, dst, src, zeros_wb)


def _final_body(is_first, u0_ref, u1_ref, d0_ref, d1_ref, s_ref, hi_ref,
                g_ref, b_ref, out_ref, hi_out_ref):
    us = u0_ref[...] + u1_ref[...]
    den = d0_ref[...] + d1_ref[...]
    agg = us / (den + 1e-16)
    t = jax.nn.relu(agg + s_ref[...])
    if is_first:
        hi_out_ref[...] = t
    else:
        t = t + hi_ref[...]
        hi_out_ref[...] = hi_ref[...]
    mu = jnp.mean(t, axis=1, keepdims=True)
    var = jnp.mean(jnp.square(t - mu), axis=1, keepdims=True)
    out_ref[...] = (t - mu) * lax.rsqrt(var + 1e-5) * g_ref[...] + b_ref[...]


def _tc_final(u, den, s, h_init, gamma, beta, is_first):
    R = 400
    grid = (N // R,)
    row = pl.BlockSpec((R, D), lambda i: (i, 0))
    col = pl.BlockSpec((R, 1), lambda i: (i, 0))
    vec = pl.BlockSpec((1, D), lambda i: (0, 0))
    return pl.pallas_call(
        functools.partial(_final_body, is_first),
        grid=grid,
        in_specs=[row, row, col, col, row, row, vec, vec],
        out_specs=[row, row],
        out_shape=[
            jax.ShapeDtypeStruct((N, D), jnp.float32),
            jax.ShapeDtypeStruct((N, D), jnp.float32),
        ],
    )(u[0], u[1], den[0], den[1], s, h_init, gamma.reshape(1, D),
      beta.reshape(1, D))


def kernel(x, edge_index, Wq, bq, Wk, bk, Wv, bv, Ws, bs, gamma, beta):
    src = edge_index[0]
    dst = edge_index[1]
    zeros_wb = jnp.zeros((WB, D), jnp.float32)

    h = x
    h_init = h
    for i in range(L):
        q, kv, s = _tc_qkvs(h, Wq[i], Wk[i], Wv[i], Ws[i], bq[i], bk[i],
                            bv[i], bs[i])
        u, dacc = _sc_fused(q, kv, dst, src, zeros_wb)
        un = u[:, :N]
        den = dacc.reshape(NC, NP, 1)[:, :N]
        h, h_init = _tc_final(un, den, s, h_init, gamma[i], beta[i], i == 0)
    return h
